# Initial kernel scaffold; baseline (speedup 1.0000x reference)
import jax
import jax.numpy as jnp
from jax.experimental import pallas as pl


def kernel(x, table):
    raise NotImplementedError("write your pallas kernel here")


# speedup vs baseline: 2.7724x; 2.7724x over previous
import functools

import jax
import jax.numpy as jnp
from jax import lax
from jax.experimental import pallas as pl
from jax.experimental.pallas import tpu as pltpu
from jax.experimental.pallas import tpu_sc as plsc

BATCH = 16384
HIST = 50
DIM = 64
NUM_CORES = 2
NUM_SUBCORES = 16
NUM_WORKERS = NUM_CORES * NUM_SUBCORES
ROWS_PER_W = BATCH // NUM_WORKERS
CHUNK = 128
NCHUNK = ROWS_PER_W // CHUNK


def _sc_body(table_hbm, idx_hbm, out_hbm, idx_v, acc_v, sem):
    wid = lax.axis_index("s") * NUM_CORES + lax.axis_index("c")
    pltpu.sync_copy(idx_hbm.at[wid], idx_v)

    for s in range(NCHUNK):
        pltpu.async_copy(
            table_hbm.at[idx_v.at[0, s]],
            acc_v.at[pl.ds(s * CHUNK, CHUNK)],
            sem,
        )
    for s in range(NCHUNK):
        pltpu.make_async_copy(
            table_hbm.at[idx_v.at[0, s]],
            acc_v.at[pl.ds(s * CHUNK, CHUNK)],
            sem,
        ).wait()

    @pl.loop(1, HIST)
    def _(l):
        for s in range(NCHUNK):
            pltpu.async_copy(
                table_hbm.at[idx_v.at[l, s]],
                acc_v.at[pl.ds(s * CHUNK, CHUNK)],
                sem,
                add=True,
            )
        for s in range(NCHUNK):
            pltpu.make_async_copy(
                table_hbm.at[idx_v.at[l, s]],
                acc_v.at[pl.ds(s * CHUNK, CHUNK)],
                sem,
            ).wait()

    pltpu.sync_copy(acc_v, out_hbm.at[pl.ds(wid * ROWS_PER_W, ROWS_PER_W)])


@functools.partial(
    pl.kernel,
    out_type=jax.ShapeDtypeStruct((BATCH, DIM), jnp.float32),
    mesh=plsc.VectorSubcoreMesh(
        core_axis_name="c", subcore_axis_name="s",
        num_cores=NUM_CORES, num_subcores=NUM_SUBCORES,
    ),
    scratch_types=[
        pltpu.VMEM((HIST, NCHUNK, CHUNK), jnp.int32),
        pltpu.VMEM((ROWS_PER_W, DIM), jnp.float32),
        pltpu.SemaphoreType.DMA,
    ],
    compiler_params=pltpu.CompilerParams(use_tc_tiling_on_sc=False),
)
def _sc_embed_sum(table_hbm, idx_hbm, out_hbm, idx_v, acc_v, sem):
    _sc_body(table_hbm, idx_hbm, out_hbm, idx_v, acc_v, sem)


def kernel(x, table):
    idx = x.T.reshape(HIST, NUM_WORKERS, NCHUNK, CHUNK).transpose(1, 0, 2, 3)
    return _sc_embed_sum(table, idx)


# speedup vs baseline: 2.8829x; 1.0398x over previous
import functools

import jax
import jax.numpy as jnp
from jax import lax
from jax.experimental import pallas as pl
from jax.experimental.pallas import tpu as pltpu
from jax.experimental.pallas import tpu_sc as plsc

BATCH = 16384
HIST = 50
DIM = 64
NUM_CORES = 2
NUM_SUBCORES = 16
NUM_WORKERS = NUM_CORES * NUM_SUBCORES
ROWS_PER_W = BATCH // NUM_WORKERS
CHUNK = 128
NCHUNK = ROWS_PER_W // CHUNK


def _sc_body(table_hbm, idx_hbm, out_hbm, idx_v, acc_v, sem):
    wid = lax.axis_index("s") * NUM_CORES + lax.axis_index("c")
    pltpu.sync_copy(idx_hbm.at[wid], idx_v)

    for s in range(NCHUNK):
        pltpu.async_copy(
            table_hbm.at[idx_v.at[0, s]],
            acc_v.at[pl.ds(s * CHUNK, CHUNK)],
            sem,
        )
    for s in range(NCHUNK):
        pltpu.make_async_copy(
            table_hbm.at[idx_v.at[0, s]],
            acc_v.at[pl.ds(s * CHUNK, CHUNK)],
            sem,
        ).wait()

    @pl.loop(1, HIST)
    def _(l):
        for s in range(NCHUNK):
            pltpu.async_copy(
                table_hbm.at[idx_v.at[l, s]],
                acc_v.at[pl.ds(s * CHUNK, CHUNK)],
                sem,
                add=True,
            )

    @pl.loop(1, HIST)
    def _(l):
        for s in range(NCHUNK):
            pltpu.make_async_copy(
                table_hbm.at[idx_v.at[l, s]],
                acc_v.at[pl.ds(s * CHUNK, CHUNK)],
                sem,
            ).wait()

    pltpu.sync_copy(acc_v, out_hbm.at[pl.ds(wid * ROWS_PER_W, ROWS_PER_W)])


@functools.partial(
    pl.kernel,
    out_type=jax.ShapeDtypeStruct((BATCH, DIM), jnp.float32),
    mesh=plsc.VectorSubcoreMesh(
        core_axis_name="c", subcore_axis_name="s",
        num_cores=NUM_CORES, num_subcores=NUM_SUBCORES,
    ),
    scratch_types=[
        pltpu.VMEM((HIST, NCHUNK, CHUNK), jnp.int32),
        pltpu.VMEM((ROWS_PER_W, DIM), jnp.float32),
        pltpu.SemaphoreType.DMA,
    ],
    compiler_params=pltpu.CompilerParams(use_tc_tiling_on_sc=False),
)
def _sc_embed_sum(table_hbm, idx_hbm, out_hbm, idx_v, acc_v, sem):
    _sc_body(table_hbm, idx_hbm, out_hbm, idx_v, acc_v, sem)


def kernel(x, table):
    idx = x.T.reshape(HIST, NUM_WORKERS, NCHUNK, CHUNK).transpose(1, 0, 2, 3)
    return _sc_embed_sum(table, idx)
